# baseline (speedup 1.0000x reference)
import jax
import jax.numpy as jnp
from jax.experimental import pallas as pl


def _xw_block(x_ref, w_ref, y_ref):
    y_ref[0, 0] = jnp.dot(
        x_ref[0], w_ref[0], preferred_element_type=jnp.float32
    ).astype(jnp.bfloat16)


def _acc_block(t_ref, y_ref, b_ref, o_ref):
    k = pl.program_id(0)
    u = pl.program_id(1)

    @pl.when((k == 0) & (u == 0))
    def _init():
        o_ref[...] = jnp.broadcast_to(b_ref[...], o_ref.shape)

    t = t_ref[0].astype(jnp.bfloat16)
    n_batch = o_ref.shape[0]
    for b in range(n_batch):
        o_ref[b] = o_ref[b] + jnp.dot(
            t, y_ref[0, b], preferred_element_type=jnp.float32
        )


@jax.jit
def kernel(input, T_k, weight, bias):
    B, V, D_IN = input.shape
    K, _, D_OUT = weight.shape
    TILE_U = min(256, V)

    x16 = input.astype(jnp.bfloat16)
    w16 = weight.astype(jnp.bfloat16)
    bias2d = bias.reshape(1, D_OUT)

    y = pl.pallas_call(
        _xw_block,
        grid=(B, K),
        in_specs=[
            pl.BlockSpec((1, V, D_IN), lambda b, k: (b, 0, 0)),
            pl.BlockSpec((1, D_IN, D_OUT), lambda b, k: (k, 0, 0)),
        ],
        out_specs=pl.BlockSpec((1, 1, V, D_OUT), lambda b, k: (k, b, 0, 0)),
        out_shape=jax.ShapeDtypeStruct((K, B, V, D_OUT), jnp.bfloat16),
    )(x16, w16)

    out = pl.pallas_call(
        _acc_block,
        grid=(K, V // TILE_U),
        in_specs=[
            pl.BlockSpec((1, V, TILE_U), lambda k, u: (k, 0, u)),
            pl.BlockSpec((1, B, TILE_U, D_OUT), lambda k, u: (k, 0, u, 0)),
            pl.BlockSpec((1, D_OUT), lambda k, u: (0, 0)),
        ],
        out_specs=pl.BlockSpec((B, V, D_OUT), lambda k, u: (0, 0, 0)),
        out_shape=jax.ShapeDtypeStruct((B, V, D_OUT), jnp.float32),
    )(T_k, y, bias2d)
    return out


# speedup vs baseline: 1.0928x; 1.0928x over previous
import jax
import jax.numpy as jnp
from jax.experimental import pallas as pl
from jax.experimental.pallas import tpu as pltpu


def _gcn_block(x_ref, t_ref, w_ref, b_ref, o_ref):
    k = pl.program_id(1)

    @pl.when(k == 0)
    def _init():
        o_ref[...] = jnp.broadcast_to(b_ref[...], o_ref.shape)

    t = t_ref[0].astype(jnp.bfloat16)
    w = w_ref[0]
    n_batch = x_ref.shape[0]
    for b in range(n_batch):
        temp = jnp.dot(t, x_ref[b], preferred_element_type=jnp.float32)
        part = jnp.dot(temp.astype(jnp.bfloat16), w,
                       preferred_element_type=jnp.float32)
        o_ref[b] = o_ref[b] + part


@jax.jit
def kernel(input, T_k, weight, bias):
    B, V, D_IN = input.shape
    K, _, D_OUT = weight.shape
    TILE_V = min(512, V)

    x16 = input.astype(jnp.bfloat16)
    w16 = weight.astype(jnp.bfloat16)
    bias2d = bias.reshape(1, D_OUT)

    out = pl.pallas_call(
        _gcn_block,
        grid=(V // TILE_V, K),
        in_specs=[
            pl.BlockSpec((B, V, D_IN), lambda i, k: (0, 0, 0)),
            pl.BlockSpec((1, TILE_V, V), lambda i, k: (k, i, 0)),
            pl.BlockSpec((1, D_IN, D_OUT), lambda i, k: (k, 0, 0)),
            pl.BlockSpec((1, D_OUT), lambda i, k: (0, 0)),
        ],
        out_specs=pl.BlockSpec((B, TILE_V, D_OUT), lambda i, k: (0, i, 0)),
        out_shape=jax.ShapeDtypeStruct((B, V, D_OUT), jnp.float32),
        compiler_params=pltpu.CompilerParams(
            vmem_limit_bytes=66 * 1024 * 1024,
        ),
    )(x16, T_k, w16, bias2d)
    return out


# speedup vs baseline: 1.3652x; 1.2492x over previous
import functools

import jax
import jax.numpy as jnp
from jax.experimental import pallas as pl
from jax.experimental.pallas import tpu as pltpu
from jax.sharding import PartitionSpec as P


_N_DEV = min(2, jax.device_count())
if _N_DEV > 1:
    _MESH = jax.make_mesh(
        (_N_DEV,), ("d",),
        axis_types=(jax.sharding.AxisType.Explicit,),
    )
    jax.sharding.set_mesh(_MESH)
else:
    _MESH = None


def _gcn_block(x_ref, t_ref, w_ref, b_ref, o_ref):
    k = pl.program_id(1)

    @pl.when(k == 0)
    def _init():
        o_ref[...] = jnp.broadcast_to(b_ref[...], o_ref.shape)

    t = t_ref[0].astype(jnp.bfloat16)
    w = w_ref[0]
    n_batch = x_ref.shape[0]
    for b in range(n_batch):
        temp = jnp.dot(t, x_ref[b], preferred_element_type=jnp.float32)
        part = jnp.dot(temp.astype(jnp.bfloat16), w,
                       preferred_element_type=jnp.float32)
        o_ref[b] = o_ref[b] + part


def _local_gcn(input, T_loc, weight, bias):
    B, V, D_IN = input.shape
    K, V_loc, _ = T_loc.shape
    D_OUT = weight.shape[-1]
    TILE_V = min(512, V_loc)

    x16 = input.astype(jnp.bfloat16)
    w16 = weight.astype(jnp.bfloat16)
    bias2d = bias.reshape(1, D_OUT)

    return pl.pallas_call(
        _gcn_block,
        grid=(V_loc // TILE_V, K),
        in_specs=[
            pl.BlockSpec((B, V, D_IN), lambda i, k: (0, 0, 0)),
            pl.BlockSpec((1, TILE_V, V), lambda i, k: (k, i, 0)),
            pl.BlockSpec((1, D_IN, D_OUT), lambda i, k: (k, 0, 0)),
            pl.BlockSpec((1, D_OUT), lambda i, k: (0, 0)),
        ],
        out_specs=pl.BlockSpec((B, TILE_V, D_OUT), lambda i, k: (0, i, 0)),
        out_shape=jax.ShapeDtypeStruct((B, V_loc, D_OUT), jnp.float32),
        compiler_params=pltpu.CompilerParams(
            vmem_limit_bytes=60 * 1024 * 1024,
        ),
    )(x16, T_loc, w16, bias2d)


@jax.jit
def kernel(input, T_k, weight, bias):
    if _MESH is not None and T_k.shape[1] % _N_DEV == 0:
        input = jax.reshard(input, P())
        T_k = jax.reshard(T_k, P(None, "d", None))
        weight = jax.reshard(weight, P())
        bias = jax.reshard(bias, P())
        fn = jax.shard_map(
            _local_gcn,
            mesh=_MESH,
            in_specs=(P(), P(None, "d", None), P(), P()),
            out_specs=P(None, "d", None),
            check_vma=False,
        )
        return fn(input, T_k, weight, bias)
    return _local_gcn(input, T_k, weight, bias)
